# baseline (speedup 1.0000x reference)
import functools

import jax
import jax.numpy as jnp
from jax import lax
from jax.experimental import pallas as pl
from jax.experimental.pallas import tpu as pltpu
from jax.experimental.pallas import tpu_sc as plsc

SAMPLES = 16
ELEMS = 512 * 512
HALF = ELEMS // 2
CHUNK = 16384
NCHUNK = HALF // CHUNK
ITERS = CHUNK // 16
MBITS = 5
NEXP = 16
NBINS = NEXP << MBITS
SHIFT = 23 - MBITS
OFFSET = 119 << MBITS
SLOTS = NBINS + 16
PLANE = SLOTS * 16
HSIZE = 2 * PLANE
HSMALL = 2 * SLOTS
UNROLL = 32


def _sc_body(logf, tgtf, hist_out, buf_l0, buf_t0, buf_l1, buf_t1,
             hcnt, hneg, hsmall, sl0, st0, sl1, st1):
    s = lax.axis_index("s")
    c = lax.axis_index("c")
    wid = s * 2 + c
    lane = lax.iota(jnp.int32, 16)
    zero16 = jnp.zeros((16,), jnp.float32)
    ones16 = jnp.full((16,), 1.0, jnp.float32)
    lane_adj = (
        lax.bitcast_convert_type(lane - OFFSET * 16, jnp.uint32),
        lax.bitcast_convert_type(lane - OFFSET * 16 + PLANE, jnp.uint32),
    )

    bufs = ((buf_l0, buf_t0, sl0, st0), (buf_l1, buf_t1, sl1, st1))
    ROWS = CHUNK // 512

    def start(ci, parity):
        bl, bt, sl, st = bufs[parity]
        row = c * (HALF // 512) + ci * ROWS
        hl = pltpu.async_copy(logf.at[s, pl.ds(row, ROWS), :], bl, sl)
        ht = pltpu.async_copy(tgtf.at[s, pl.ds(row, ROWS), :], bt, st)
        return hl, ht

    handles = [None, None]
    handles[0] = start(0, 0)

    def z_body(i, carry):
        for u in range(8):
            hcnt[pl.ds(i * 128 + u * 16, 16)] = zero16
            hneg[pl.ds(i * 128 + u * 16, 16)] = zero16
        return carry

    lax.fori_loop(0, HSIZE // 128, z_body, jnp.int32(0))
    for ci in range(NCHUNK):
        par = ci % 2
        if ci + 1 < NCHUNK:
            handles[1 - par] = start(ci + 1, 1 - par)
        hl, ht = handles[par]
        hl.wait()
        ht.wait()
        bl, bt = bufs[par][0], bufs[par][1]

        def inner(j, pc, bl=bl, bt=bt):
            vals = []
            gpr_log2 = {1: 0, 2: 1, 4: 2, 8: 3, 16: 4, 32: 5}[32 // UNROLL]
            r = j >> gpr_log2
            cg = (j & ((32 // UNROLL) - 1)) * (UNROLL * 16)
            for u in range(UNROLL):
                o = cg + u * 16
                lg = bl[r, pl.ds(o, 16)]
                tg = bt[r, pl.ds(o, 16)]
                m_t = tg != 0
                e = 1.0 - jnp.where(m_t, lg, 0.0 - lg)
                bu = lax.bitcast_convert_type(e, jnp.uint32) >> SHIFT
                bu = jnp.minimum(jnp.maximum(bu, jnp.uint32(OFFSET)),
                                 jnp.uint32(OFFSET + NBINS))
                addr = lax.bitcast_convert_type(
                    bu * 16 + lane_adj[u % 2], jnp.int32)
                vals.append((addr, jnp.logical_not(m_t)))
            for addr, m_nt in vals:
                plsc.addupdate_scatter(hcnt, [addr], ones16)
                plsc.addupdate_scatter(hneg, [addr], ones16, mask=m_nt)
            return pc

        lax.fori_loop(0, ITERS // UNROLL, inner, jnp.int32(0))

    def red_body(g, carry):
        b16 = (g * 16 + lane) * 16
        acc_c = zero16
        acc_n = zero16
        for cp in (0, PLANE):
            for l in range(16):
                acc_c = acc_c + plsc.load_gather(hcnt, [b16 + cp + l])
                acc_n = acc_n + plsc.load_gather(hneg, [b16 + cp + l])
        hsmall[pl.ds(g * 16, 16)] = acc_c
        hsmall[pl.ds(SLOTS + g * 16, 16)] = acc_n
        return carry

    lax.fori_loop(0, SLOTS // 16, red_body, jnp.int32(0))

    pltpu.sync_copy(hsmall, hist_out.at[pl.ds(wid * HSMALL, HSMALL)])


@functools.cache
def _get_sc_hist():
    return functools.partial(
        pl.kernel,
        mesh=plsc.VectorSubcoreMesh(core_axis_name="c", subcore_axis_name="s"),
        compiler_params=pltpu.CompilerParams(needs_layout_passes=False),
        out_type=jax.ShapeDtypeStruct((32 * HSMALL,), jnp.float32),
        scratch_types=[
            pltpu.VMEM((CHUNK // 512, 512), jnp.float32),
            pltpu.VMEM((CHUNK // 512, 512), jnp.int32),
            pltpu.VMEM((CHUNK // 512, 512), jnp.float32),
            pltpu.VMEM((CHUNK // 512, 512), jnp.int32),
            pltpu.VMEM((HSIZE,), jnp.float32),
            pltpu.VMEM((HSIZE,), jnp.float32),
            pltpu.VMEM((HSMALL,), jnp.float32),
            pltpu.SemaphoreType.DMA,
            pltpu.SemaphoreType.DMA,
            pltpu.SemaphoreType.DMA,
            pltpu.SemaphoreType.DMA,
        ],
    )(_sc_body)


def _tc_body(h_ref, o_ref):
    h = h_ref[:]
    h2 = h[:, 0] + h[:, 1]
    cnt = h2[:, 0, :NBINS]
    neg = h2[:, 1, :NBINS]
    ptot = ELEMS - jnp.sum(h2[:, 1], axis=1, keepdims=True)

    def cum(a):
        sft = 1
        while sft < NBINS:
            a = a + jnp.concatenate(
                [jnp.zeros((SAMPLES, sft), jnp.float32), a[:, :-sft]], axis=1)
            sft *= 2
        return a

    ic = cum(cnt)
    inm = cum(neg)
    n_gt = ic[:, NBINS - 1:] - ic
    m_gt = inm[:, NBINS - 1:] - inm
    k = lax.broadcasted_iota(jnp.int32, (SAMPLES, NBINS), 1)
    a_lo = lax.bitcast_convert_type((k + OFFSET) << SHIFT, jnp.float32)
    a_lo = jnp.where(k == 0, 0.0, a_lo)
    a_hi = lax.bitcast_convert_type((k + 1 + OFFSET) << SHIFT, jnp.float32)
    hw = a_hi - a_lo
    num = hw * (n_gt + 0.5 * cnt)
    den = ptot + m_gt + 0.5 * neg
    contrib = jnp.where(den > 0.0, num / den, 0.0)
    o_ref[...] = jnp.sum(contrib, keepdims=True) / SAMPLES


def kernel(logits, targets):
    hist_out = _get_sc_hist()(logits, targets)
    h4 = hist_out.reshape(SAMPLES, 2, 2, SLOTS)
    out = pl.pallas_call(
        _tc_body,
        out_shape=jax.ShapeDtypeStruct((1, 1), jnp.float32),
    )(h4)
    return out[0, 0]


# speedup vs baseline: 1.0180x; 1.0180x over previous
import functools

import jax
import jax.numpy as jnp
from jax import lax
from jax.experimental import pallas as pl
from jax.experimental.pallas import tpu as pltpu
from jax.experimental.pallas import tpu_sc as plsc

SAMPLES = 16
ELEMS = 512 * 512
HALF = ELEMS // 2
CHUNK = 16384
NCHUNK = HALF // CHUNK
ITERS = CHUNK // 16
MBITS = 5
NEXP = 16
NBINS = NEXP << MBITS
SHIFT = 23 - MBITS
OFFSET = 119 << MBITS
SLOTS = NBINS + 16
PLANE = SLOTS * 16
HSIZE = 2 * PLANE
HSMALL = 2 * SLOTS
UNROLL = 16


def _sc_body(logf, tgtf, hist_out, buf_l0, buf_t0, buf_l1, buf_t1,
             hcnt, hneg, hsmall, sl0, st0, sl1, st1):
    s = lax.axis_index("s")
    c = lax.axis_index("c")
    wid = s * 2 + c
    lane = lax.iota(jnp.int32, 16)
    zero16 = jnp.zeros((16,), jnp.float32)
    ones16 = jnp.full((16,), 1.0, jnp.float32)
    lane_adj = (
        lax.bitcast_convert_type(lane - OFFSET * 16, jnp.uint32),
        lax.bitcast_convert_type(lane - OFFSET * 16 + PLANE, jnp.uint32),
    )

    bufs = ((buf_l0, buf_t0, sl0, st0), (buf_l1, buf_t1, sl1, st1))
    ROWS = CHUNK // 512

    def start(ci, parity):
        bl, bt, sl, st = bufs[parity]
        row = c * (HALF // 512) + ci * ROWS
        hl = pltpu.async_copy(logf.at[s, pl.ds(row, ROWS), :], bl, sl)
        ht = pltpu.async_copy(tgtf.at[s, pl.ds(row, ROWS), :], bt, st)
        return hl, ht

    handles = [None, None]
    handles[0] = start(0, 0)

    def z_body(i, carry):
        for u in range(8):
            hcnt[pl.ds(i * 128 + u * 16, 16)] = zero16
            hneg[pl.ds(i * 128 + u * 16, 16)] = zero16
        return carry

    lax.fori_loop(0, HSIZE // 128, z_body, jnp.int32(0))
    for ci in range(NCHUNK):
        par = ci % 2
        if ci + 1 < NCHUNK:
            handles[1 - par] = start(ci + 1, 1 - par)
        hl, ht = handles[par]
        hl.wait()
        ht.wait()
        bl, bt = bufs[par][0], bufs[par][1]

        def inner(j, pc, bl=bl, bt=bt):
            vals = []
            gpr_log2 = {1: 0, 2: 1, 4: 2, 8: 3, 16: 4, 32: 5}[32 // UNROLL]
            r = j >> gpr_log2
            cg = (j & ((32 // UNROLL) - 1)) * (UNROLL * 16)
            for u in range(UNROLL):
                o = cg + u * 16
                lg = bl[r, pl.ds(o, 16)]
                tg = bt[r, pl.ds(o, 16)]
                m_t = tg != 0
                e = 1.0 - jnp.where(m_t, lg, 0.0 - lg)
                bu = lax.bitcast_convert_type(e, jnp.uint32) >> SHIFT
                bu = jnp.minimum(jnp.maximum(bu, jnp.uint32(OFFSET)),
                                 jnp.uint32(OFFSET + NBINS))
                addr = lax.bitcast_convert_type(
                    bu * 16 + lane_adj[u % 2], jnp.int32)
                vals.append((addr, jnp.logical_not(m_t)))
            for addr, m_nt in vals:
                plsc.addupdate_scatter(hcnt, [addr], ones16)
                plsc.addupdate_scatter(hneg, [addr], ones16, mask=m_nt)
            return pc

        lax.fori_loop(0, ITERS // UNROLL, inner, jnp.int32(0))

    def red_body(g, carry):
        b16 = (g * 16 + lane) * 16
        acc_c = zero16
        acc_n = zero16
        for cp in (0, PLANE):
            for l in range(16):
                acc_c = acc_c + plsc.load_gather(hcnt, [b16 + cp + l])
                acc_n = acc_n + plsc.load_gather(hneg, [b16 + cp + l])
        hsmall[pl.ds(g * 16, 16)] = acc_c
        hsmall[pl.ds(SLOTS + g * 16, 16)] = acc_n
        return carry

    lax.fori_loop(0, SLOTS // 16, red_body, jnp.int32(0))

    pltpu.sync_copy(hsmall, hist_out.at[pl.ds(wid * HSMALL, HSMALL)])


@functools.cache
def _get_sc_hist():
    return functools.partial(
        pl.kernel,
        mesh=plsc.VectorSubcoreMesh(core_axis_name="c", subcore_axis_name="s"),
        compiler_params=pltpu.CompilerParams(needs_layout_passes=False),
        out_type=jax.ShapeDtypeStruct((32 * HSMALL,), jnp.float32),
        scratch_types=[
            pltpu.VMEM((CHUNK // 512, 512), jnp.float32),
            pltpu.VMEM((CHUNK // 512, 512), jnp.int32),
            pltpu.VMEM((CHUNK // 512, 512), jnp.float32),
            pltpu.VMEM((CHUNK // 512, 512), jnp.int32),
            pltpu.VMEM((HSIZE,), jnp.float32),
            pltpu.VMEM((HSIZE,), jnp.float32),
            pltpu.VMEM((HSMALL,), jnp.float32),
            pltpu.SemaphoreType.DMA,
            pltpu.SemaphoreType.DMA,
            pltpu.SemaphoreType.DMA,
            pltpu.SemaphoreType.DMA,
        ],
    )(_sc_body)


def _tc_body(h_ref, o_ref):
    h = h_ref[:]
    h2 = h[:, 0] + h[:, 1]
    cnt = h2[:, 0, :NBINS]
    neg = h2[:, 1, :NBINS]
    ptot = ELEMS - jnp.sum(h2[:, 1], axis=1, keepdims=True)

    def cum(a):
        sft = 1
        while sft < NBINS:
            a = a + jnp.concatenate(
                [jnp.zeros((SAMPLES, sft), jnp.float32), a[:, :-sft]], axis=1)
            sft *= 2
        return a

    ic = cum(cnt)
    inm = cum(neg)
    n_gt = ic[:, NBINS - 1:] - ic
    m_gt = inm[:, NBINS - 1:] - inm
    k = lax.broadcasted_iota(jnp.int32, (SAMPLES, NBINS), 1)
    a_lo = lax.bitcast_convert_type((k + OFFSET) << SHIFT, jnp.float32)
    a_lo = jnp.where(k == 0, 0.0, a_lo)
    a_hi = lax.bitcast_convert_type((k + 1 + OFFSET) << SHIFT, jnp.float32)
    hw = a_hi - a_lo
    num = hw * (n_gt + 0.5 * cnt)
    den = ptot + m_gt + 0.5 * neg
    contrib = jnp.where(den > 0.0, num / den, 0.0)
    o_ref[...] = jnp.sum(contrib, keepdims=True) / SAMPLES


def kernel(logits, targets):
    hist_out = _get_sc_hist()(logits, targets)
    h4 = hist_out.reshape(SAMPLES, 2, 2, SLOTS)
    out = pl.pallas_call(
        _tc_body,
        out_shape=jax.ShapeDtypeStruct((1, 1), jnp.float32),
    )(h4)
    return out[0, 0]
